# baseline (speedup 1.0000x reference)
import functools

import jax
import jax.numpy as jnp
from jax import lax
from jax.experimental import pallas as pl
from jax.experimental.pallas import tpu as pltpu
from jax.experimental.pallas import tpu_sc as plsc

N = 10000
E = 320000
IN_FEAT = 128
HIDDEN = 64
OUT_FEAT = 64
GRID = 4

NC = 2
NS = 16
NTILES = NC * NS
BATCH = 512
NBAT = E // BATCH
NB = 20
NB_LO = NBAT // NTILES
NB_REM = NBAT - NB_LO * NTILES
ACC_ROWS = N + 16
RPT = N // NS

ROW_BLK = 2000
NBLK = N // ROW_BLK
EBLK = E // NBLK


def _lin_in_body(x_ref, e_ref, w_ref, b_ref, h_ref, r_ref, c_ref):
    acc = lax.dot_general(
        x_ref[...], w_ref[...], (((1,), (1,)), ((), ())),
        preferred_element_type=jnp.float32)
    h_ref[...] = acc + b_ref[...]

    @pl.when(pl.program_id(0) == 0)
    def _():
        r_ref[...] = e_ref[0]
        c_ref[...] = e_ref[1]


def _lin_in(x, edge_index, W_in, b_in):
    return pl.pallas_call(
        _lin_in_body,
        grid=(NBLK,),
        in_specs=[
            pl.BlockSpec((ROW_BLK, IN_FEAT), lambda i: (i, 0)),
            pl.BlockSpec((2, E), lambda i: (0, 0)),
            pl.BlockSpec((HIDDEN, IN_FEAT), lambda i: (0, 0)),
            pl.BlockSpec((1, HIDDEN), lambda i: (0, 0)),
        ],
        out_specs=[
            pl.BlockSpec((ROW_BLK, HIDDEN), lambda i: (i, 0)),
            pl.BlockSpec((E,), lambda i: (0,)),
            pl.BlockSpec((E,), lambda i: (0,)),
        ],
        out_shape=[
            jax.ShapeDtypeStruct((N, HIDDEN), jnp.float32),
            jax.ShapeDtypeStruct((E,), jnp.int32),
            jax.ShapeDtypeStruct((E,), jnp.int32),
        ],
    )(x, edge_index, W_in, b_in.reshape(1, HIDDEN))


def _sc_spmm_body(rows_hbm, cols_hbm, h_hbm, z_hbm, dumr_hbm, dumc_hbm,
                  out_hbm, colf, rowf, rba, rbb, acc, semga, semgb):
    c = lax.axis_index("c")
    s = lax.axis_index("s")
    wid = c * NS + s

    pltpu.sync_copy(z_hbm, acc.at[pl.ds(s * RPT, RPT)])

    start = NB_LO * wid + jnp.minimum(wid, NB_REM)
    e0 = start * BATCH
    nreal = NB_LO * BATCH
    pltpu.sync_copy(rows_hbm.at[pl.ds(e0, nreal)], rowf.at[pl.ds(0, nreal)])
    pltpu.sync_copy(cols_hbm.at[pl.ds(e0, nreal)], colf.at[pl.ds(0, nreal)])

    @pl.when(wid < NB_REM)
    def _():
        pltpu.sync_copy(rows_hbm.at[pl.ds(e0 + nreal, BATCH)],
                        rowf.at[pl.ds(nreal, BATCH)])
        pltpu.sync_copy(cols_hbm.at[pl.ds(e0 + nreal, BATCH)],
                        colf.at[pl.ds(nreal, BATCH)])

    @pl.when(wid >= NB_REM)
    def _():
        pltpu.sync_copy(dumr_hbm, rowf.at[pl.ds(nreal, BATCH)])
        pltpu.sync_copy(dumc_hbm, colf.at[pl.ds(nreal, BATCH)])

    plsc.subcore_barrier()

    pltpu.async_copy(h_hbm.at[colf.at[pl.ds(0, BATCH)]], rba, semga)

    def body(j2, carry):
        j = j2 * 2
        ia = pl.ds(j * BATCH, BATCH)
        ib = pl.ds((j + 1) * BATCH, BATCH)
        ic = pl.ds((j + 2) * BATCH, BATCH)
        pltpu.make_async_copy(h_hbm.at[colf.at[ia]], rba, semga).wait()
        pltpu.async_copy(h_hbm.at[colf.at[ib]], rbb, semgb)
        pltpu.sync_copy(rba, acc.at[rowf.at[ia]], add=True)
        pltpu.make_async_copy(h_hbm.at[colf.at[ib]], rbb, semgb).wait()
        pltpu.async_copy(h_hbm.at[colf.at[ic]], rba, semga)
        pltpu.sync_copy(rbb, acc.at[rowf.at[ib]], add=True)
        return carry

    lax.fori_loop(0, NB // 2 - 1, body, 0)
    ia = pl.ds((NB - 2) * BATCH, BATCH)
    ib = pl.ds((NB - 1) * BATCH, BATCH)
    pltpu.make_async_copy(h_hbm.at[colf.at[ia]], rba, semga).wait()
    pltpu.async_copy(h_hbm.at[colf.at[ib]], rbb, semgb)
    pltpu.sync_copy(rba, acc.at[rowf.at[ia]], add=True)
    pltpu.make_async_copy(h_hbm.at[colf.at[ib]], rbb, semgb).wait()
    pltpu.sync_copy(rbb, acc.at[rowf.at[ib]], add=True)
    plsc.subcore_barrier()

    pltpu.sync_copy(acc.at[pl.ds(s * RPT, RPT)],
                    out_hbm.at[pl.ds(c * N + s * RPT, RPT), pl.ds(0, HIDDEN)])


_sc_spmm = functools.partial(
    pl.kernel,
    out_type=jax.ShapeDtypeStruct((2 * N, 128), jnp.float32),
    mesh=plsc.VectorSubcoreMesh(
        core_axis_name="c", subcore_axis_name="s",
        num_cores=NC, num_subcores=NS),
    scratch_types=[
        pltpu.VMEM((NB * BATCH,), jnp.int32),
        pltpu.VMEM((NB * BATCH,), jnp.int32),
        pltpu.VMEM((BATCH, HIDDEN), jnp.float32),
        pltpu.VMEM((BATCH, HIDDEN), jnp.float32),
        pltpu.VMEM_SHARED((ACC_ROWS, HIDDEN), jnp.float32),
        pltpu.SemaphoreType.DMA,
        pltpu.SemaphoreType.DMA,
    ],
    compiler_params=pltpu.CompilerParams(use_tc_tiling_on_sc=False),
)(_sc_spmm_body)


def _post_body(p0_ref, p1_ref, wc_ref, ws_ref, wo_ref, o_ref):
    a = p0_ref[:, :HIDDEN] + p1_ref[:, :HIDDEN]
    c1 = jnp.cos(a)
    s1 = jnp.sin(a)
    ck, sk = c1, s1
    y = lax.dot_general(c1, wc_ref[0], (((1,), (0,)), ((), ())),
                        preferred_element_type=jnp.float32)
    y = y + lax.dot_general(s1, ws_ref[0], (((1,), (0,)), ((), ())),
                            preferred_element_type=jnp.float32)
    for g in range(1, GRID):
        ck, sk = ck * c1 - sk * s1, sk * c1 + ck * s1
        y = y + lax.dot_general(ck, wc_ref[g], (((1,), (0,)), ((), ())),
                                preferred_element_type=jnp.float32)
        y = y + lax.dot_general(sk, ws_ref[g], (((1,), (0,)), ((), ())),
                                preferred_element_type=jnp.float32)
    o = lax.dot_general(y, wo_ref[...], (((1,), (1,)), ((), ())),
                        preferred_element_type=jnp.float32)
    m = jnp.max(o, axis=-1, keepdims=True)
    ex = jnp.exp(o - m)
    o_ref[...] = (o - m) - jnp.log(jnp.sum(ex, axis=-1, keepdims=True))


def _post(partials, Wc, Ws, W_out):
    return pl.pallas_call(
        _post_body,
        grid=(NBLK,),
        in_specs=[
            pl.BlockSpec((ROW_BLK, 128), lambda i: (i, 0)),
            pl.BlockSpec((ROW_BLK, 128), lambda i: (NBLK + i, 0)),
            pl.BlockSpec((GRID, HIDDEN, HIDDEN), lambda i: (0, 0, 0)),
            pl.BlockSpec((GRID, HIDDEN, HIDDEN), lambda i: (0, 0, 0)),
            pl.BlockSpec((OUT_FEAT, HIDDEN), lambda i: (0, 0)),
        ],
        out_specs=pl.BlockSpec((ROW_BLK, OUT_FEAT), lambda i: (i, 0)),
        out_shape=jax.ShapeDtypeStruct((N, OUT_FEAT), jnp.float32),
    )(partials, partials, Wc, Ws, W_out)


def kernel(x, edge_index, W_in, b_in, coeffs0, W_out):
    h, rows1d, cols1d = _lin_in(x, edge_index, W_in, b_in)
    zeros = jnp.zeros((RPT, HIDDEN), jnp.float32)
    dummy_rows = jnp.full((BATCH,), N, jnp.int32)
    dummy_cols = jnp.zeros((BATCH,), jnp.int32)
    partials = _sc_spmm(rows1d, cols1d, h, zeros, dummy_rows, dummy_cols)
    Wc = jnp.transpose(coeffs0[0], (2, 1, 0))
    Ws = jnp.transpose(coeffs0[1], (2, 1, 0))
    return _post(partials, Wc, Ws, W_out)


# speedup vs baseline: 1.0037x; 1.0037x over previous
import functools

import jax
import jax.numpy as jnp
from jax import lax
from jax.experimental import pallas as pl
from jax.experimental.pallas import tpu as pltpu
from jax.experimental.pallas import tpu_sc as plsc

N = 10000
E = 320000
IN_FEAT = 128
HIDDEN = 64
OUT_FEAT = 64
GRID = 4

NC = 2
NS = 16
NTILES = NC * NS
BATCH = 512
NBAT = E // BATCH
NB = 20
NB_LO = NBAT // NTILES
NB_REM = NBAT - NB_LO * NTILES
ACC_ROWS = N + BATCH
RPT = N // NS

ROW_BLK = 2000
NBLK = N // ROW_BLK
EBLK = E // NBLK


def _lin_in_body(x_ref, e_ref, w_ref, b_ref, h_ref, r_ref, c_ref):
    acc = lax.dot_general(
        x_ref[...], w_ref[...], (((1,), (1,)), ((), ())),
        preferred_element_type=jnp.float32)
    h_ref[...] = acc + b_ref[...]

    @pl.when(pl.program_id(0) == 0)
    def _():
        r_ref[...] = e_ref[0]
        c_ref[...] = e_ref[1]


def _lin_in(x, edge_index, W_in, b_in):
    return pl.pallas_call(
        _lin_in_body,
        grid=(NBLK,),
        in_specs=[
            pl.BlockSpec((ROW_BLK, IN_FEAT), lambda i: (i, 0)),
            pl.BlockSpec((2, E), lambda i: (0, 0)),
            pl.BlockSpec((HIDDEN, IN_FEAT), lambda i: (0, 0)),
            pl.BlockSpec((1, HIDDEN), lambda i: (0, 0)),
        ],
        out_specs=[
            pl.BlockSpec((ROW_BLK, HIDDEN), lambda i: (i, 0)),
            pl.BlockSpec((E,), lambda i: (0,)),
            pl.BlockSpec((E,), lambda i: (0,)),
        ],
        out_shape=[
            jax.ShapeDtypeStruct((N, HIDDEN), jnp.float32),
            jax.ShapeDtypeStruct((E,), jnp.int32),
            jax.ShapeDtypeStruct((E,), jnp.int32),
        ],
    )(x, edge_index, W_in, b_in.reshape(1, HIDDEN))


def _sc_spmm_body(rows_hbm, cols_hbm, h_hbm, z_hbm, dumr_hbm, dumc_hbm,
                  out_hbm, colf, rowf, rba, rbb, acc, semga, semgb):
    c = lax.axis_index("c")
    s = lax.axis_index("s")
    wid = c * NS + s

    pltpu.sync_copy(z_hbm, acc.at[pl.ds(s * RPT, RPT)])

    start = NB_LO * wid + jnp.minimum(wid, NB_REM)
    e0 = start * BATCH
    nreal = NB_LO * BATCH
    pltpu.sync_copy(rows_hbm.at[pl.ds(e0, nreal)], rowf.at[pl.ds(0, nreal)])
    pltpu.sync_copy(cols_hbm.at[pl.ds(e0, nreal)], colf.at[pl.ds(0, nreal)])

    @pl.when(wid < NB_REM)
    def _():
        pltpu.sync_copy(rows_hbm.at[pl.ds(e0 + nreal, BATCH)],
                        rowf.at[pl.ds(nreal, BATCH)])
        pltpu.sync_copy(cols_hbm.at[pl.ds(e0 + nreal, BATCH)],
                        colf.at[pl.ds(nreal, BATCH)])

    @pl.when(wid >= NB_REM)
    def _():
        pltpu.sync_copy(dumr_hbm, rowf.at[pl.ds(nreal, BATCH)])
        pltpu.sync_copy(dumc_hbm, colf.at[pl.ds(nreal, BATCH)])

    plsc.subcore_barrier()

    pltpu.async_copy(h_hbm.at[colf.at[pl.ds(0, BATCH)]], rba, semga)

    def body(j2, carry):
        j = j2 * 2
        ia = pl.ds(j * BATCH, BATCH)
        ib = pl.ds((j + 1) * BATCH, BATCH)
        ic = pl.ds((j + 2) * BATCH, BATCH)
        pltpu.make_async_copy(h_hbm.at[colf.at[ia]], rba, semga).wait()
        pltpu.async_copy(h_hbm.at[colf.at[ib]], rbb, semgb)
        pltpu.sync_copy(rba, acc.at[rowf.at[ia]], add=True)
        pltpu.make_async_copy(h_hbm.at[colf.at[ib]], rbb, semgb).wait()
        pltpu.async_copy(h_hbm.at[colf.at[ic]], rba, semga)
        pltpu.sync_copy(rbb, acc.at[rowf.at[ib]], add=True)
        return carry

    lax.fori_loop(0, NB // 2 - 1, body, 0)
    ia = pl.ds((NB - 2) * BATCH, BATCH)
    ib = pl.ds((NB - 1) * BATCH, BATCH)
    pltpu.make_async_copy(h_hbm.at[colf.at[ia]], rba, semga).wait()
    pltpu.async_copy(h_hbm.at[colf.at[ib]], rbb, semgb)
    pltpu.sync_copy(rba, acc.at[rowf.at[ia]], add=True)
    pltpu.make_async_copy(h_hbm.at[colf.at[ib]], rbb, semgb).wait()
    pltpu.sync_copy(rbb, acc.at[rowf.at[ib]], add=True)
    plsc.subcore_barrier()

    pltpu.sync_copy(acc.at[pl.ds(s * RPT, RPT)],
                    out_hbm.at[pl.ds(c * N + s * RPT, RPT), pl.ds(0, HIDDEN)])


_sc_spmm = functools.partial(
    pl.kernel,
    out_type=jax.ShapeDtypeStruct((2 * N, 128), jnp.float32),
    mesh=plsc.VectorSubcoreMesh(
        core_axis_name="c", subcore_axis_name="s",
        num_cores=NC, num_subcores=NS),
    scratch_types=[
        pltpu.VMEM((NB * BATCH,), jnp.int32),
        pltpu.VMEM((NB * BATCH,), jnp.int32),
        pltpu.VMEM((BATCH, HIDDEN), jnp.float32),
        pltpu.VMEM((BATCH, HIDDEN), jnp.float32),
        pltpu.VMEM_SHARED((ACC_ROWS, HIDDEN), jnp.float32),
        pltpu.SemaphoreType.DMA,
        pltpu.SemaphoreType.DMA,
    ],
    compiler_params=pltpu.CompilerParams(use_tc_tiling_on_sc=False),
)(_sc_spmm_body)


def _post_body(p0_ref, p1_ref, wc_ref, ws_ref, wo_ref, o_ref):
    a = p0_ref[:, :HIDDEN] + p1_ref[:, :HIDDEN]
    c1 = jnp.cos(a)
    s1 = jnp.sin(a)
    ck, sk = c1, s1
    y = lax.dot_general(c1, wc_ref[0], (((1,), (0,)), ((), ())),
                        preferred_element_type=jnp.float32)
    y = y + lax.dot_general(s1, ws_ref[0], (((1,), (0,)), ((), ())),
                            preferred_element_type=jnp.float32)
    for g in range(1, GRID):
        ck, sk = ck * c1 - sk * s1, sk * c1 + ck * s1
        y = y + lax.dot_general(ck, wc_ref[g], (((1,), (0,)), ((), ())),
                                preferred_element_type=jnp.float32)
        y = y + lax.dot_general(sk, ws_ref[g], (((1,), (0,)), ((), ())),
                                preferred_element_type=jnp.float32)
    o = lax.dot_general(y, wo_ref[...], (((1,), (1,)), ((), ())),
                        preferred_element_type=jnp.float32)
    m = jnp.max(o, axis=-1, keepdims=True)
    ex = jnp.exp(o - m)
    o_ref[...] = (o - m) - jnp.log(jnp.sum(ex, axis=-1, keepdims=True))


def _post(partials, Wc, Ws, W_out):
    return pl.pallas_call(
        _post_body,
        grid=(NBLK,),
        in_specs=[
            pl.BlockSpec((ROW_BLK, 128), lambda i: (i, 0)),
            pl.BlockSpec((ROW_BLK, 128), lambda i: (NBLK + i, 0)),
            pl.BlockSpec((GRID, HIDDEN, HIDDEN), lambda i: (0, 0, 0)),
            pl.BlockSpec((GRID, HIDDEN, HIDDEN), lambda i: (0, 0, 0)),
            pl.BlockSpec((OUT_FEAT, HIDDEN), lambda i: (0, 0)),
        ],
        out_specs=pl.BlockSpec((ROW_BLK, OUT_FEAT), lambda i: (i, 0)),
        out_shape=jax.ShapeDtypeStruct((N, OUT_FEAT), jnp.float32),
    )(partials, partials, Wc, Ws, W_out)


def kernel(x, edge_index, W_in, b_in, coeffs0, W_out):
    h, rows1d, cols1d = _lin_in(x, edge_index, W_in, b_in)
    zeros = jnp.zeros((RPT, HIDDEN), jnp.float32)
    dummy_rows = N + jnp.arange(BATCH, dtype=jnp.int32)
    dummy_cols = jnp.zeros((BATCH,), jnp.int32)
    partials = _sc_spmm(rows1d, cols1d, h, zeros, dummy_rows, dummy_cols)
    Wc = jnp.transpose(coeffs0[0], (2, 1, 0))
    Ws = jnp.transpose(coeffs0[1], (2, 1, 0))
    return _post(partials, Wc, Ws, W_out)


# speedup vs baseline: 2.1177x; 2.1099x over previous
import functools

import jax
import jax.numpy as jnp
from jax import lax
from jax.experimental import pallas as pl
from jax.experimental.pallas import tpu as pltpu
from jax.experimental.pallas import tpu_sc as plsc

N = 10000
E = 320000
IN_FEAT = 128
HIDDEN = 64
OUT_FEAT = 64
GRID = 4

NC = 2
NS = 16
NTILES = NC * NS
BATCH = 512
NBAT = E // BATCH
NB = 20
NB_LO = NBAT // NTILES
NB_REM = NBAT - NB_LO * NTILES
ACC_ROWS = N + BATCH
RPT = N // NS

ROW_BLK = 2000
NBLK = N // ROW_BLK
EBLK = E // NBLK


def _lin_in_body(x_ref, e_ref, w_ref, b_ref, h_ref, r_ref, c_ref):
    acc = lax.dot_general(
        x_ref[...], w_ref[...], (((1,), (1,)), ((), ())),
        preferred_element_type=jnp.float32)
    h_ref[...] = acc + b_ref[...]

    @pl.when(pl.program_id(0) == 0)
    def _():
        r_ref[...] = e_ref[0]
        c_ref[...] = e_ref[1]


def _lin_in(x, edge_index, W_in, b_in):
    return pl.pallas_call(
        _lin_in_body,
        grid=(NBLK,),
        in_specs=[
            pl.BlockSpec((ROW_BLK, IN_FEAT), lambda i: (i, 0)),
            pl.BlockSpec((2, E), lambda i: (0, 0)),
            pl.BlockSpec((HIDDEN, IN_FEAT), lambda i: (0, 0)),
            pl.BlockSpec((1, HIDDEN), lambda i: (0, 0)),
        ],
        out_specs=[
            pl.BlockSpec((ROW_BLK, HIDDEN), lambda i: (i, 0)),
            pl.BlockSpec((E,), lambda i: (0,)),
            pl.BlockSpec((E,), lambda i: (0,)),
        ],
        out_shape=[
            jax.ShapeDtypeStruct((N, HIDDEN), jnp.float32),
            jax.ShapeDtypeStruct((E,), jnp.int32),
            jax.ShapeDtypeStruct((E,), jnp.int32),
        ],
    )(x, edge_index, W_in, b_in.reshape(1, HIDDEN))


def _sc_spmm_body(rows_hbm, cols_hbm, h_hbm, z_hbm, dumr_hbm, dumc_hbm,
                  out_hbm, colf, rowf, rba, rbb, acc, semga, semgb):
    c = lax.axis_index("c")
    s = lax.axis_index("s")
    wid = c * NS + s

    pltpu.sync_copy(z_hbm, acc.at[pl.ds(s * RPT, RPT)])

    start = NB_LO * wid + jnp.minimum(wid, NB_REM)
    e0 = start * BATCH
    nreal = NB_LO * BATCH
    pltpu.sync_copy(rows_hbm.at[pl.ds(e0, nreal)], rowf.at[pl.ds(0, nreal)])
    pltpu.sync_copy(cols_hbm.at[pl.ds(e0, nreal)], colf.at[pl.ds(0, nreal)])

    @pl.when(wid < NB_REM)
    def _():
        pltpu.sync_copy(rows_hbm.at[pl.ds(e0 + nreal, BATCH)],
                        rowf.at[pl.ds(nreal, BATCH)])
        pltpu.sync_copy(cols_hbm.at[pl.ds(e0 + nreal, BATCH)],
                        colf.at[pl.ds(nreal, BATCH)])

    @pl.when(wid >= NB_REM)
    def _():
        pltpu.sync_copy(dumr_hbm, rowf.at[pl.ds(nreal, BATCH)])
        pltpu.sync_copy(dumc_hbm, colf.at[pl.ds(nreal, BATCH)])

    plsc.subcore_barrier()

    pltpu.async_copy(h_hbm.at[colf.at[pl.ds(0, BATCH)]], rba, semga)

    def body(j2, carry):
        j = j2 * 2
        ia = pl.ds(j * BATCH, BATCH)
        ib = pl.ds((j + 1) * BATCH, BATCH)
        ic = pl.ds((j + 2) * BATCH, BATCH)
        pltpu.make_async_copy(h_hbm.at[colf.at[ia]], rba, semga).wait()
        pltpu.async_copy(h_hbm.at[colf.at[ib]], rbb, semgb)
        pltpu.sync_copy(rba, acc.at[rowf.at[ia]], add=True)
        pltpu.make_async_copy(h_hbm.at[colf.at[ib]], rbb, semgb).wait()
        pltpu.async_copy(h_hbm.at[colf.at[ic]], rba, semga)
        pltpu.sync_copy(rbb, acc.at[rowf.at[ib]], add=True)
        return carry

    lax.fori_loop(0, NB // 2 - 1, body, 0)
    ia = pl.ds((NB - 2) * BATCH, BATCH)
    ib = pl.ds((NB - 1) * BATCH, BATCH)
    pltpu.make_async_copy(h_hbm.at[colf.at[ia]], rba, semga).wait()
    pltpu.async_copy(h_hbm.at[colf.at[ib]], rbb, semgb)
    pltpu.sync_copy(rba, acc.at[rowf.at[ia]], add=True)
    pltpu.make_async_copy(h_hbm.at[colf.at[ib]], rbb, semgb).wait()
    pltpu.sync_copy(rbb, acc.at[rowf.at[ib]], add=True)
    plsc.subcore_barrier()

    pltpu.sync_copy(acc.at[pl.ds(s * RPT, RPT)],
                    out_hbm.at[pl.ds(c * N + s * RPT, RPT), pl.ds(0, HIDDEN)])


_sc_spmm = functools.partial(
    pl.kernel,
    out_type=jax.ShapeDtypeStruct((2 * N, 128), jnp.float32),
    mesh=plsc.VectorSubcoreMesh(
        core_axis_name="c", subcore_axis_name="s",
        num_cores=NC, num_subcores=NS),
    scratch_types=[
        pltpu.VMEM((NB * BATCH,), jnp.int32),
        pltpu.VMEM((NB * BATCH,), jnp.int32),
        pltpu.VMEM((BATCH, HIDDEN), jnp.float32),
        pltpu.VMEM((BATCH, HIDDEN), jnp.float32),
        pltpu.VMEM_SHARED((ACC_ROWS, HIDDEN), jnp.float32),
        pltpu.SemaphoreType.DMA,
        pltpu.SemaphoreType.DMA,
    ],
    compiler_params=pltpu.CompilerParams(use_tc_tiling_on_sc=False),
)(_sc_spmm_body)


def _post_body(p0_ref, p1_ref, wc_ref, ws_ref, wo_ref, o_ref):
    a = p0_ref[:, :HIDDEN] + p1_ref[:, :HIDDEN]
    c1 = jnp.cos(a)
    s1 = jnp.sin(a)
    ck, sk = c1, s1
    y = lax.dot_general(c1, wc_ref[0], (((1,), (0,)), ((), ())),
                        preferred_element_type=jnp.float32)
    y = y + lax.dot_general(s1, ws_ref[0], (((1,), (0,)), ((), ())),
                            preferred_element_type=jnp.float32)
    for g in range(1, GRID):
        ck, sk = ck * c1 - sk * s1, sk * c1 + ck * s1
        y = y + lax.dot_general(ck, wc_ref[g], (((1,), (0,)), ((), ())),
                                preferred_element_type=jnp.float32)
        y = y + lax.dot_general(sk, ws_ref[g], (((1,), (0,)), ((), ())),
                                preferred_element_type=jnp.float32)
    o = lax.dot_general(y, wo_ref[...], (((1,), (1,)), ((), ())),
                        preferred_element_type=jnp.float32)
    m = jnp.max(o, axis=-1, keepdims=True)
    ex = jnp.exp(o - m)
    o_ref[...] = (o - m) - jnp.log(jnp.sum(ex, axis=-1, keepdims=True))


def _post(partials, Wc, Ws, W_out):
    return pl.pallas_call(
        _post_body,
        grid=(NBLK,),
        in_specs=[
            pl.BlockSpec((ROW_BLK, 128), lambda i: (i, 0)),
            pl.BlockSpec((ROW_BLK, 128), lambda i: (NBLK + i, 0)),
            pl.BlockSpec((GRID, HIDDEN, HIDDEN), lambda i: (0, 0, 0)),
            pl.BlockSpec((GRID, HIDDEN, HIDDEN), lambda i: (0, 0, 0)),
            pl.BlockSpec((OUT_FEAT, HIDDEN), lambda i: (0, 0)),
        ],
        out_specs=pl.BlockSpec((ROW_BLK, OUT_FEAT), lambda i: (i, 0)),
        out_shape=jax.ShapeDtypeStruct((N, OUT_FEAT), jnp.float32),
    )(partials, partials, Wc, Ws, W_out)


def kernel(x, edge_index, W_in, b_in, coeffs0, W_out):
    h, rows1d, cols1d = _lin_in(x, edge_index, W_in, b_in)
    zeros = jnp.zeros((RPT, HIDDEN), jnp.float32)
    dummy_rows = N + jnp.arange(BATCH, dtype=jnp.int32)
    dummy_cols = jnp.arange(BATCH, dtype=jnp.int32)
    partials = _sc_spmm(rows1d, cols1d, h, zeros, dummy_rows, dummy_cols)
    Wc = jnp.transpose(coeffs0[0], (2, 1, 0))
    Ws = jnp.transpose(coeffs0[1], (2, 1, 0))
    return _post(partials, Wc, Ws, W_out)


# speedup vs baseline: 2.3320x; 1.1012x over previous
import functools

import jax
import jax.numpy as jnp
from jax import lax
from jax.experimental import pallas as pl
from jax.experimental.pallas import tpu as pltpu
from jax.experimental.pallas import tpu_sc as plsc

N = 10000
E = 320000
IN_FEAT = 128
HIDDEN = 64
OUT_FEAT = 64
GRID = 4

NC = 2
NS = 16
NTILES = NC * NS
BATCH = 512
NBAT = E // BATCH
NB = 20
NB_LO = NBAT // NTILES
NB_REM = NBAT - NB_LO * NTILES
ACC_ROWS = N + BATCH
RPT = N // NS

ROW_BLK = 2000
NBLK = N // ROW_BLK
EBLK = E // NBLK


def _lin_in_body(x_ref, e_ref, w_ref, b_ref, h_ref, r_ref, c_ref):
    acc = lax.dot_general(
        x_ref[...], w_ref[...], (((1,), (1,)), ((), ())),
        preferred_element_type=jnp.float32)
    h_ref[...] = acc + b_ref[...]

    @pl.when(pl.program_id(0) == 0)
    def _():
        r_ref[...] = e_ref[0]
        c_ref[...] = e_ref[1]


def _lin_in(x, edge_index, W_in, b_in):
    return pl.pallas_call(
        _lin_in_body,
        grid=(NBLK,),
        in_specs=[
            pl.BlockSpec((ROW_BLK, IN_FEAT), lambda i: (i, 0)),
            pl.BlockSpec((2, E), lambda i: (0, 0)),
            pl.BlockSpec((HIDDEN, IN_FEAT), lambda i: (0, 0)),
            pl.BlockSpec((1, HIDDEN), lambda i: (0, 0)),
        ],
        out_specs=[
            pl.BlockSpec((ROW_BLK, HIDDEN), lambda i: (i, 0)),
            pl.BlockSpec((E,), lambda i: (0,)),
            pl.BlockSpec((E,), lambda i: (0,)),
        ],
        out_shape=[
            jax.ShapeDtypeStruct((N, HIDDEN), jnp.float32),
            jax.ShapeDtypeStruct((E,), jnp.int32),
            jax.ShapeDtypeStruct((E,), jnp.int32),
        ],
    )(x, edge_index, W_in, b_in.reshape(1, HIDDEN))


def _sc_spmm_body(rows_hbm, cols_hbm, h_hbm, z_hbm, dumr_hbm, dumc_hbm,
                  out_hbm, colf, rowf, rba, rbb, acc, semga, semgb):
    c = lax.axis_index("c")
    s = lax.axis_index("s")
    wid = c * NS + s

    pltpu.sync_copy(z_hbm, acc.at[pl.ds(s * RPT, RPT)])

    start = NB_LO * wid + jnp.minimum(wid, NB_REM)
    e0 = start * BATCH
    nreal = NB_LO * BATCH
    pltpu.sync_copy(rows_hbm.at[pl.ds(e0, nreal)], rowf.at[pl.ds(0, nreal)])
    pltpu.sync_copy(cols_hbm.at[pl.ds(e0, nreal)], colf.at[pl.ds(0, nreal)])

    @pl.when(wid < NB_REM)
    def _():
        pltpu.sync_copy(rows_hbm.at[pl.ds(e0 + nreal, BATCH)],
                        rowf.at[pl.ds(nreal, BATCH)])
        pltpu.sync_copy(cols_hbm.at[pl.ds(e0 + nreal, BATCH)],
                        colf.at[pl.ds(nreal, BATCH)])

    @pl.when(wid >= NB_REM)
    def _():
        pltpu.sync_copy(dumr_hbm, rowf.at[pl.ds(nreal, BATCH)])
        pltpu.sync_copy(dumc_hbm, colf.at[pl.ds(nreal, BATCH)])

    plsc.subcore_barrier()

    pltpu.async_copy(h_hbm.at[colf.at[pl.ds(0, BATCH)]], rba, semga)

    def body(j2, carry):
        j = j2 * 2
        ia = pl.ds(j * BATCH, BATCH)
        ib = pl.ds((j + 1) * BATCH, BATCH)
        ic = pl.ds((j + 2) * BATCH, BATCH)
        pltpu.make_async_copy(h_hbm.at[colf.at[ia]], rba, semga).wait()
        pltpu.async_copy(h_hbm.at[colf.at[ib]], rbb, semgb)
        pltpu.sync_copy(rba, acc.at[rowf.at[ia]], add=True)
        pltpu.make_async_copy(h_hbm.at[colf.at[ib]], rbb, semgb).wait()
        pltpu.async_copy(h_hbm.at[colf.at[ic]], rba, semga)
        pltpu.sync_copy(rbb, acc.at[rowf.at[ib]], add=True)
        return carry

    lax.fori_loop(0, NB // 2 - 1, body, 0)
    ia = pl.ds((NB - 2) * BATCH, BATCH)
    ib = pl.ds((NB - 1) * BATCH, BATCH)
    pltpu.make_async_copy(h_hbm.at[colf.at[ia]], rba, semga).wait()
    pltpu.async_copy(h_hbm.at[colf.at[ib]], rbb, semgb)
    pltpu.sync_copy(rba, acc.at[rowf.at[ia]], add=True)
    pltpu.make_async_copy(h_hbm.at[colf.at[ib]], rbb, semgb).wait()
    pltpu.sync_copy(rbb, acc.at[rowf.at[ib]], add=True)
    plsc.subcore_barrier()

    pltpu.sync_copy(acc.at[pl.ds(s * RPT, RPT)],
                    out_hbm.at[pl.ds(c * N + s * RPT, RPT), pl.ds(0, HIDDEN)])


_sc_spmm = functools.partial(
    pl.kernel,
    out_type=jax.ShapeDtypeStruct((2 * N, 128), jnp.float32),
    mesh=plsc.VectorSubcoreMesh(
        core_axis_name="c", subcore_axis_name="s",
        num_cores=NC, num_subcores=NS),
    scratch_types=[
        pltpu.VMEM((NB * BATCH,), jnp.int32),
        pltpu.VMEM((NB * BATCH,), jnp.int32),
        pltpu.VMEM((BATCH, HIDDEN), jnp.float32),
        pltpu.VMEM((BATCH, HIDDEN), jnp.float32),
        pltpu.VMEM_SHARED((ACC_ROWS, HIDDEN), jnp.float32),
        pltpu.SemaphoreType.DMA,
        pltpu.SemaphoreType.DMA,
    ],
    compiler_params=pltpu.CompilerParams(use_tc_tiling_on_sc=False),
)(_sc_spmm_body)


def _post_body(p0_ref, p1_ref, wc_ref, ws_ref, wo_ref, o_ref):
    a = p0_ref[:, :HIDDEN] + p1_ref[:, :HIDDEN]
    r = a - 6.283185307179586 * jnp.round(a * 0.15915494309189535)
    r2 = r * r
    s1 = r * (1 + r2 * (-1 / 6 + r2 * (1 / 120 + r2 * (-1 / 5040 + r2 * (
        1 / 362880 + r2 * (-1 / 39916800 + r2 * (1 / 6227020800)))))))
    c1 = 1 + r2 * (-1 / 2 + r2 * (1 / 24 + r2 * (-1 / 720 + r2 * (
        1 / 40320 + r2 * (-1 / 3628800 + r2 * (
            1 / 479001600 + r2 * (-1 / 87178291200)))))))
    ck, sk = c1, s1
    y = lax.dot_general(c1, wc_ref[0], (((1,), (0,)), ((), ())),
                        preferred_element_type=jnp.float32)
    y = y + lax.dot_general(s1, ws_ref[0], (((1,), (0,)), ((), ())),
                            preferred_element_type=jnp.float32)
    for g in range(1, GRID):
        ck, sk = ck * c1 - sk * s1, sk * c1 + ck * s1
        y = y + lax.dot_general(ck, wc_ref[g], (((1,), (0,)), ((), ())),
                                preferred_element_type=jnp.float32)
        y = y + lax.dot_general(sk, ws_ref[g], (((1,), (0,)), ((), ())),
                                preferred_element_type=jnp.float32)
    o = lax.dot_general(y, wo_ref[...], (((1,), (1,)), ((), ())),
                        preferred_element_type=jnp.float32)
    m = jnp.max(o, axis=-1, keepdims=True)
    ex = jnp.exp(o - m)
    o_ref[...] = (o - m) - jnp.log(jnp.sum(ex, axis=-1, keepdims=True))


def _post(partials, Wc, Ws, W_out):
    return pl.pallas_call(
        _post_body,
        grid=(NBLK,),
        in_specs=[
            pl.BlockSpec((ROW_BLK, 128), lambda i: (i, 0)),
            pl.BlockSpec((ROW_BLK, 128), lambda i: (NBLK + i, 0)),
            pl.BlockSpec((GRID, HIDDEN, HIDDEN), lambda i: (0, 0, 0)),
            pl.BlockSpec((GRID, HIDDEN, HIDDEN), lambda i: (0, 0, 0)),
            pl.BlockSpec((OUT_FEAT, HIDDEN), lambda i: (0, 0)),
        ],
        out_specs=pl.BlockSpec((ROW_BLK, OUT_FEAT), lambda i: (i, 0)),
        out_shape=jax.ShapeDtypeStruct((N, OUT_FEAT), jnp.float32),
    )(partials, partials, Wc, Ws, W_out)


def kernel(x, edge_index, W_in, b_in, coeffs0, W_out):
    h, rows1d, cols1d = _lin_in(x, edge_index, W_in, b_in)
    zeros = jnp.zeros((RPT, HIDDEN), jnp.float32)
    dummy_rows = N + jnp.arange(BATCH, dtype=jnp.int32)
    dummy_cols = jnp.arange(BATCH, dtype=jnp.int32)
    partials = _sc_spmm(rows1d, cols1d, h, zeros, dummy_rows, dummy_cols)
    Wc = jnp.transpose(coeffs0[0], (2, 1, 0))
    Ws = jnp.transpose(coeffs0[1], (2, 1, 0))
    return _post(partials, Wc, Ws, W_out)
